# baseline (speedup 1.0000x reference)
import functools

import jax
import jax.numpy as jnp
from jax import lax
from jax.experimental import pallas as pl
from jax.experimental.pallas import tpu as pltpu
from jax.experimental.pallas import tpu_sc as plsc

N_NODES = 10000
N_EDGES = 320000
F = 128
L = 16
NC, NS = 2, 16
NW = NC * NS
CHUNK = 64
NROWS = N_EDGES // CHUNK
CPW = 160
CPW_LAST = NROWS - (NW - 1) * CPW
BLK = 8
NBUF = 4
N_PAD = 10240
ROWS_PER_TILE = N_PAD // NS


def _splat_lane(vec16, i):
    idx = jnp.full((L, 1), i, jnp.int32)
    dn = lax.GatherDimensionNumbers(
        offset_dims=(), collapsed_slice_dims=(0,), start_index_map=(0,)
    )
    return lax.gather(
        vec16, idx, dn, slice_sizes=(1,),
        mode=lax.GatherScatterMode.PROMISE_IN_BOUNDS,
    )


def _sc_aggregate(x, row2, col2, vals2):
    mesh = plsc.VectorSubcoreMesh(
        core_axis_name="c", subcore_axis_name="s", num_cores=NC, num_subcores=NS
    )

    @functools.partial(
        pl.kernel,
        out_type=jax.ShapeDtypeStruct((NC, N_PAD, F), jnp.float32),
        mesh=mesh,
        scratch_types=[
            pltpu.VMEM_SHARED((N_PAD, F), jnp.float32),
            pltpu.VMEM((2, BLK, CHUNK), jnp.int32),
            pltpu.VMEM((2, BLK, CHUNK), jnp.int32),
            pltpu.VMEM((2, BLK, CHUNK), jnp.float32),
        ]
        + [pltpu.VMEM((CHUNK, F), jnp.float32) for _ in range(NBUF + 1)]
        + [pltpu.SemaphoreType.DMA for _ in range(NBUF + 2)],
    )
    def agg(x_hbm, row_hbm, col_hbm, vals_hbm, out_hbm,
            acc, colstg, rowstg, valstg, *bufs_sems):
        gbuf = list(bufs_sems[0:NBUF])
        sbuf = bufs_sems[NBUF]
        gsem = list(bufs_sems[NBUF + 1:2 * NBUF + 1])
        ssem = bufs_sems[2 * NBUF + 1]
        isem = bufs_sems[2 * NBUF + 2]
        c = lax.axis_index("c")
        s = lax.axis_index("s")
        wid = s * NC + c
        stripe = pl.ds(s * ROWS_PER_TILE, ROWS_PER_TILE)
        r0 = wid * CPW
        nblk = jnp.where(wid == NW - 1, CPW_LAST // BLK, CPW // BLK)
        nchunks = nblk * BLK

        pltpu.sync_copy(col_hbm.at[pl.ds(r0, BLK)], colstg.at[0])
        pltpu.sync_copy(row_hbm.at[pl.ds(r0, BLK)], rowstg.at[0])
        pltpu.sync_copy(vals_hbm.at[pl.ds(r0, BLK)], valstg.at[0])

        @plsc.parallel_loop(0, CHUNK)
        def zero_body(r):
            for jz in range(F // L):
                sbuf[r, pl.ds(jz * L, L)] = jnp.zeros((L,), jnp.float32)

        for kz in range(ROWS_PER_TILE // CHUNK):
            pltpu.sync_copy(
                sbuf, acc.at[pl.ds(s * ROWS_PER_TILE + kz * CHUNK, CHUNK)])
        plsc.subcore_barrier()

        def wait_gather(b):
            pltpu.make_async_copy(
                x_hbm.at[colstg.at[0].at[0]], gbuf[b], gsem[b]
            ).wait()

        def wait_scatter():
            pltpu.make_async_copy(
                sbuf, acc.at[rowstg.at[0].at[0]], ssem
            ).wait()

        def wait_stage():
            for ref in (colstg, rowstg, valstg):
                pltpu.make_async_copy(
                    col_hbm.at[pl.ds(0, BLK)], ref.at[0], isem
                ).wait()

        def scale_chunk(m, j, gb):
            @plsc.parallel_loop(0, CHUNK // L, unroll=4)
            def group_body(g):
                e0 = g * L
                vals16 = valstg[m, j, pl.ds(e0, L)]
                for ii in range(L):
                    val = _splat_lane(vals16, ii)
                    for jj in range(F // L):
                        sl = pl.ds(jj * L, L)
                        sbuf[e0 + ii, sl] = gb[e0 + ii, sl] * val

        pltpu.async_copy(x_hbm.at[colstg.at[0].at[0]], gbuf[0], gsem[0])
        pltpu.async_copy(x_hbm.at[colstg.at[0].at[1]], gbuf[1], gsem[1])
        pltpu.async_copy(x_hbm.at[colstg.at[0].at[2]], gbuf[2], gsem[2])

        def block_body(p, carry):
            m = lax.rem(p, 2)
            m2 = 1 - m
            has_next = p + 1 < nblk
            for j in range(BLK):
                i = p * BLK + j
                b = j % NBUF
                wait_gather(b)

                if j == 0:

                    @pl.when(p > 0)
                    def _():
                        wait_scatter()
                else:
                    wait_scatter()

                scale_chunk(m, j, gbuf[b])
                pltpu.async_copy(
                    sbuf, acc.at[rowstg.at[m].at[j]], ssem, add=True
                )

                b3 = (j + 3) % NBUF
                if j < BLK - 3:
                    colref = colstg.at[m].at[j + 3]
                else:
                    colref = colstg.at[m2].at[j + 3 - BLK]
                if j == BLK - 3:

                    @pl.when(has_next)
                    def _():
                        wait_stage()

                @pl.when(i + 3 < nchunks)
                def _():
                    pltpu.async_copy(x_hbm.at[colref], gbuf[b3], gsem[b3])

                if j == 1:
                    @pl.when(has_next)
                    def _():
                        rnxt = r0 + (p + 1) * BLK
                        pltpu.async_copy(
                            col_hbm.at[pl.ds(rnxt, BLK)], colstg.at[m2], isem)
                        pltpu.async_copy(
                            row_hbm.at[pl.ds(rnxt, BLK)], rowstg.at[m2], isem)
                        pltpu.async_copy(
                            vals_hbm.at[pl.ds(rnxt, BLK)], valstg.at[m2], isem)
            return carry

        lax.fori_loop(0, nblk, block_body, 0)

        wait_scatter()

        plsc.subcore_barrier()
        pltpu.sync_copy(acc.at[stripe], out_hbm.at[c].at[stripe])

    return agg(x, row2, col2, vals2)


ROW_BLK = 1000


def _tc_xw_body(x_ref, w_ref, b_ref, o_ref):
    o_ref[...] = (
        jnp.dot(x_ref[...], w_ref[...], preferred_element_type=jnp.float32)
        + b_ref[...]
    )


def _tc_xw(x, weight, bias):
    grid = (N_NODES // ROW_BLK,)
    return pl.pallas_call(
        _tc_xw_body,
        grid=grid,
        in_specs=[
            pl.BlockSpec((ROW_BLK, F), lambda i: (i, 0)),
            pl.BlockSpec((F, F), lambda i: (0, 0)),
            pl.BlockSpec((1, F), lambda i: (0, 0)),
        ],
        out_specs=pl.BlockSpec((ROW_BLK, F), lambda i: (i, 0)),
        out_shape=jax.ShapeDtypeStruct((N_NODES, F), jnp.float32),
    )(x, weight, bias)


def _tc_body(y_ref, p_ref, scale_ref, w_ref, o_ref):
    e1s = (p_ref[0] + p_ref[1]) * scale_ref[...]
    o_ref[...] = y_ref[...] - jnp.dot(
        e1s, w_ref[...], preferred_element_type=jnp.float32
    )


def _tc_finish(y, partials, scale, weight):
    grid = (N_NODES // ROW_BLK,)
    return pl.pallas_call(
        _tc_body,
        grid=grid,
        in_specs=[
            pl.BlockSpec((ROW_BLK, F), lambda i: (i, 0)),
            pl.BlockSpec((NC, ROW_BLK, F), lambda i: (0, i, 0)),
            pl.BlockSpec((1, F), lambda i: (0, 0)),
            pl.BlockSpec((F, F), lambda i: (0, 0)),
        ],
        out_specs=pl.BlockSpec((ROW_BLK, F), lambda i: (i, 0)),
        out_shape=jax.ShapeDtypeStruct((N_NODES, F), jnp.float32),
    )(y, partials, scale, weight)


def kernel(input, edge_index, edge_vals, weight, learnable_diag_1, bias):
    row2 = edge_index[0].astype(jnp.int32).reshape(NROWS, CHUNK)
    col2 = edge_index[1].astype(jnp.int32).reshape(NROWS, CHUNK)
    vals2 = edge_vals.astype(jnp.float32).reshape(NROWS, CHUNK)
    partials = _sc_aggregate(input, row2, col2, vals2)
    y = _tc_xw(input, weight, bias.reshape(1, F))
    scale = (1.0 + learnable_diag_1).reshape(1, F).astype(jnp.float32)
    return _tc_finish(y, partials, scale, weight)


# speedup vs baseline: 1.3538x; 1.3538x over previous
import functools

import jax
import jax.numpy as jnp
from jax import lax
from jax.experimental import pallas as pl
from jax.experimental.pallas import tpu as pltpu
from jax.experimental.pallas import tpu_sc as plsc

N_NODES = 10000
N_EDGES = 320000
F = 128
L = 16
NC, NS = 2, 16
NW = NC * NS
CHUNK = 64
NROWS = N_EDGES // CHUNK
CPW = 160
CPW_LAST = NROWS - (NW - 1) * CPW
BLK = 8
NBUF = 4
N_PAD = 10240
ROWS_PER_TILE = N_PAD // NS


def _splat_lane(vec16, i):
    idx = jnp.full((L, 1), i, jnp.int32)
    dn = lax.GatherDimensionNumbers(
        offset_dims=(), collapsed_slice_dims=(0,), start_index_map=(0,)
    )
    return lax.gather(
        vec16, idx, dn, slice_sizes=(1,),
        mode=lax.GatherScatterMode.PROMISE_IN_BOUNDS,
    )


def _sc_aggregate(x, row2, col2, vals2):
    mesh = plsc.VectorSubcoreMesh(
        core_axis_name="c", subcore_axis_name="s", num_cores=NC, num_subcores=NS
    )

    @functools.partial(
        pl.kernel,
        out_type=jax.ShapeDtypeStruct((NC, N_PAD, F), jnp.float32),
        mesh=mesh,
        scratch_types=[
            pltpu.VMEM_SHARED((N_PAD, F), jnp.float32),
            pltpu.VMEM((2, BLK, CHUNK), jnp.int32),
            pltpu.VMEM((2, BLK, CHUNK), jnp.int32),
            pltpu.VMEM((2, BLK, CHUNK), jnp.float32),
        ]
        + [pltpu.VMEM((CHUNK, F), jnp.float32) for _ in range(NBUF + 1)]
        + [pltpu.SemaphoreType.DMA for _ in range(NBUF + 2)],
    )
    def agg(x_hbm, row_hbm, col_hbm, vals_hbm, out_hbm,
            acc, colstg, rowstg, valstg, *bufs_sems):
        gbuf = list(bufs_sems[0:NBUF])
        sbuf = bufs_sems[NBUF]
        gsem = list(bufs_sems[NBUF + 1:2 * NBUF + 1])
        ssem = bufs_sems[2 * NBUF + 1]
        isem = bufs_sems[2 * NBUF + 2]
        c = lax.axis_index("c")
        s = lax.axis_index("s")
        wid = s * NC + c
        stripe = pl.ds(s * ROWS_PER_TILE, ROWS_PER_TILE)
        r0 = wid * CPW
        nblk = jnp.where(wid == NW - 1, CPW_LAST // BLK, CPW // BLK)
        nchunks = nblk * BLK

        pltpu.sync_copy(col_hbm.at[pl.ds(r0, BLK)], colstg.at[0])
        pltpu.sync_copy(row_hbm.at[pl.ds(r0, BLK)], rowstg.at[0])
        pltpu.sync_copy(vals_hbm.at[pl.ds(r0, BLK)], valstg.at[0])

        @plsc.parallel_loop(0, CHUNK)
        def zero_body(r):
            for jz in range(F // L):
                sbuf[r, pl.ds(jz * L, L)] = jnp.zeros((L,), jnp.float32)

        for kz in range(ROWS_PER_TILE // CHUNK):
            pltpu.sync_copy(
                sbuf, acc.at[pl.ds(s * ROWS_PER_TILE + kz * CHUNK, CHUNK)])
        plsc.subcore_barrier()

        def wait_gather(b):
            pltpu.make_async_copy(
                x_hbm.at[colstg.at[0].at[0]], gbuf[b], gsem[b]
            ).wait()

        def wait_scatter():
            pltpu.make_async_copy(
                sbuf, acc.at[rowstg.at[0].at[0]], ssem
            ).wait()

        def wait_stage():
            for ref in (colstg, rowstg, valstg):
                pltpu.make_async_copy(
                    col_hbm.at[pl.ds(0, BLK)], ref.at[0], isem
                ).wait()

        def scale_chunk(m, j, gb):
            @plsc.parallel_loop(0, CHUNK // L, unroll=2)
            def group_body(g):
                e0 = g * L
                vals16 = valstg[m, j, pl.ds(e0, L)]
                for ii in range(L):
                    val = _splat_lane(vals16, ii)
                    for jj in range(F // L):
                        sl = pl.ds(jj * L, L)
                        sbuf[e0 + ii, sl] = gb[e0 + ii, sl] * val

        pltpu.async_copy(x_hbm.at[colstg.at[0].at[0]], gbuf[0], gsem[0])
        pltpu.async_copy(x_hbm.at[colstg.at[0].at[1]], gbuf[1], gsem[1])
        pltpu.async_copy(x_hbm.at[colstg.at[0].at[2]], gbuf[2], gsem[2])

        def block_body(p, carry):
            m = lax.rem(p, 2)
            m2 = 1 - m
            has_next = p + 1 < nblk
            for j in range(BLK):
                i = p * BLK + j
                b = j % NBUF
                wait_gather(b)

                if j == 0:

                    @pl.when(p > 0)
                    def _():
                        wait_scatter()
                else:
                    wait_scatter()

                scale_chunk(m, j, gbuf[b])
                pltpu.async_copy(
                    sbuf, acc.at[rowstg.at[m].at[j]], ssem, add=True
                )

                b3 = (j + 3) % NBUF
                if j < BLK - 3:
                    colref = colstg.at[m].at[j + 3]
                else:
                    colref = colstg.at[m2].at[j + 3 - BLK]
                if j == BLK - 3:

                    @pl.when(has_next)
                    def _():
                        wait_stage()

                @pl.when(i + 3 < nchunks)
                def _():
                    pltpu.async_copy(x_hbm.at[colref], gbuf[b3], gsem[b3])

                if j == 1:
                    @pl.when(has_next)
                    def _():
                        rnxt = r0 + (p + 1) * BLK
                        pltpu.async_copy(
                            col_hbm.at[pl.ds(rnxt, BLK)], colstg.at[m2], isem)
                        pltpu.async_copy(
                            row_hbm.at[pl.ds(rnxt, BLK)], rowstg.at[m2], isem)
                        pltpu.async_copy(
                            vals_hbm.at[pl.ds(rnxt, BLK)], valstg.at[m2], isem)
            return carry

        lax.fori_loop(0, nblk, block_body, 0)

        wait_scatter()

        plsc.subcore_barrier()
        pltpu.sync_copy(acc.at[stripe], out_hbm.at[c].at[stripe])

    return agg(x, row2, col2, vals2)


ROW_BLK = 1000


def _tc_xw_body(x_ref, w_ref, b_ref, o_ref):
    o_ref[...] = (
        jnp.dot(x_ref[...], w_ref[...], preferred_element_type=jnp.float32)
        + b_ref[...]
    )


def _tc_xw(x, weight, bias):
    grid = (N_NODES // ROW_BLK,)
    return pl.pallas_call(
        _tc_xw_body,
        grid=grid,
        in_specs=[
            pl.BlockSpec((ROW_BLK, F), lambda i: (i, 0)),
            pl.BlockSpec((F, F), lambda i: (0, 0)),
            pl.BlockSpec((1, F), lambda i: (0, 0)),
        ],
        out_specs=pl.BlockSpec((ROW_BLK, F), lambda i: (i, 0)),
        out_shape=jax.ShapeDtypeStruct((N_NODES, F), jnp.float32),
    )(x, weight, bias)


def _tc_body(y_ref, p_ref, scale_ref, w_ref, o_ref):
    e1s = (p_ref[0] + p_ref[1]) * scale_ref[...]
    o_ref[...] = y_ref[...] - jnp.dot(
        e1s, w_ref[...], preferred_element_type=jnp.float32
    )


def _tc_finish(y, partials, scale, weight):
    grid = (N_NODES // ROW_BLK,)
    return pl.pallas_call(
        _tc_body,
        grid=grid,
        in_specs=[
            pl.BlockSpec((ROW_BLK, F), lambda i: (i, 0)),
            pl.BlockSpec((NC, ROW_BLK, F), lambda i: (0, i, 0)),
            pl.BlockSpec((1, F), lambda i: (0, 0)),
            pl.BlockSpec((F, F), lambda i: (0, 0)),
        ],
        out_specs=pl.BlockSpec((ROW_BLK, F), lambda i: (i, 0)),
        out_shape=jax.ShapeDtypeStruct((N_NODES, F), jnp.float32),
    )(y, partials, scale, weight)


def kernel(input, edge_index, edge_vals, weight, learnable_diag_1, bias):
    row2 = edge_index[0].astype(jnp.int32).reshape(NROWS, CHUNK)
    col2 = edge_index[1].astype(jnp.int32).reshape(NROWS, CHUNK)
    vals2 = edge_vals.astype(jnp.float32).reshape(NROWS, CHUNK)
    partials = _sc_aggregate(input, row2, col2, vals2)
    y = _tc_xw(input, weight, bias.reshape(1, F))
    scale = (1.0 + learnable_diag_1).reshape(1, F).astype(jnp.float32)
    return _tc_finish(y, partials, scale, weight)
